# baseline (speedup 1.0000x reference)
import functools

import jax
import jax.numpy as jnp
from jax import lax
from jax.experimental import pallas as pl
from jax.experimental.pallas import tpu as pltpu
from jax.experimental.pallas import tpu_sc as plsc

DIM = 64
BATCH = 16384
NC = 2
NS = 16
L = 16
NW = NC * NS
BPW = BATCH // NW
CHUNK = 128
NCH = BPW // CHUNK


@functools.partial(
    pl.kernel,
    out_type=jax.ShapeDtypeStruct((BATCH,), jnp.float32),
    mesh=plsc.VectorSubcoreMesh(core_axis_name="c", subcore_axis_name="s"),
    compiler_params=pltpu.CompilerParams(use_tc_tiling_on_sc=False),
    scratch_types=[
        pltpu.VMEM((NCH, CHUNK), jnp.int32),
        pltpu.VMEM((NCH, CHUNK), jnp.int32),
        pltpu.VMEM((BPW, DIM), jnp.float32),
        pltpu.VMEM((BPW, DIM), jnp.float32),
        pltpu.VMEM((BPW,), jnp.float32),
        pltpu.SemaphoreType.DMA,
    ],
)
def _sgns_kernel(cs_hbm, os_hbm, wemb_hbm, bemb_hbm, out_hbm,
                 cs_idx, os_idx, cs_rows, os_rows, out_v, sem):
    wid = lax.axis_index("s") * NC + lax.axis_index("c")
    base = wid * BPW

    for k in range(NCH):
        pltpu.sync_copy(cs_hbm.at[pl.ds(base + k * CHUNK, CHUNK)], cs_idx.at[k])
        pltpu.sync_copy(os_hbm.at[pl.ds(base + k * CHUNK, CHUNK)], os_idx.at[k])

    cps = []
    for k in range(NCH):
        cps.append(pltpu.async_copy(
            wemb_hbm.at[cs_idx.at[k]], cs_rows.at[pl.ds(k * CHUNK, CHUNK)], sem))
        cps.append(pltpu.async_copy(
            bemb_hbm.at[os_idx.at[k]], os_rows.at[pl.ds(k * CHUNK, CHUNK)], sem))
    for cp in cps:
        cp.wait()

    lanes = lax.iota(jnp.int32, L)
    lane_masks = [lanes == j for j in range(L)]
    _dnums = lax.GatherDimensionNumbers(
        offset_dims=(), collapsed_slice_dims=(0,), start_index_map=(0,))

    def lane_shuffle(v, idx):
        return lax.gather(v, idx[:, None], _dnums, slice_sizes=(1,),
                          mode=lax.GatherScatterMode.PROMISE_IN_BOUNDS)

    def group(g, carry):
        out_acc = jnp.zeros((L,), jnp.float32)
        for j in range(L):
            r = g * L + j
            acc = cs_rows[r, pl.ds(0, L)] * os_rows[r, pl.ds(0, L)]
            for c in range(1, DIM // L):
                acc = acc + cs_rows[r, pl.ds(c * L, L)] * os_rows[r, pl.ds(c * L, L)]
            for sh in (8, 4, 2, 1):
                acc = acc + lane_shuffle(acc, lanes ^ sh)
            out_acc = jnp.where(lane_masks[j], acc, out_acc)
        out_v[pl.ds(g * L, L)] = 1.0 / (1.0 + jnp.exp(-out_acc))
        return carry

    lax.fori_loop(0, BPW // L, group, 0)
    pltpu.sync_copy(out_v, out_hbm.at[pl.ds(base, BPW)])


def kernel(cs, os, word_embs, bkp_word_embs):
    return _sgns_kernel(cs.astype(jnp.int32), os.astype(jnp.int32),
                        word_embs, bkp_word_embs)
